# baseline (speedup 1.0000x reference)
import jax
import jax.numpy as jnp
from jax import lax
from jax.experimental import pallas as pl
from jax.experimental.pallas import tpu as pltpu
from jax.experimental.pallas import tpu_sc as plsc

B = 16384
D = 32
NC = 2
NS = 16
L = 16
NW = NC * NS
BPW = B // NW


def _sc_body(uid_hbm, iid_hbm, ut_hbm, it_hbm, out_hbm,
             uidx_v, iidx_v, urows_v, irows_v, out_v, sem_u, sem_i):
    wid = lax.axis_index("s") * NC + lax.axis_index("c")
    base = wid * BPW
    pltpu.sync_copy(uid_hbm.at[pl.ds(base, BPW)], uidx_v)
    pltpu.sync_copy(iid_hbm.at[pl.ds(base, BPW)], iidx_v)
    cu = pltpu.async_copy(ut_hbm.at[uidx_v], urows_v, sem_u)
    ci = pltpu.async_copy(it_hbm.at[iidx_v], irows_v, sem_i)
    cu.wait()
    ci.wait()

    lanes = lax.iota(jnp.int32, L)

    def body(g, carry):
        rows = g * L + lanes
        acc = jnp.zeros((L,), jnp.float32)
        for d0 in range(D):
            dcol = (d0 + lanes) & (D - 1)
            uc = plsc.load_gather(urows_v, [rows, dcol])
            ic = plsc.load_gather(irows_v, [rows, dcol])
            acc = acc + uc * ic
        out_v[pl.ds(g * L, L)] = acc
        return carry

    lax.fori_loop(0, BPW // L, body, 0)
    pltpu.sync_copy(out_v, out_hbm.at[pl.ds(base, BPW)])


@jax.jit
def kernel(user_ids, item_ids, user_table, item_table):
    mesh = plsc.VectorSubcoreMesh(core_axis_name="c", subcore_axis_name="s")
    f = pl.kernel(
        _sc_body,
        out_type=jax.ShapeDtypeStruct((B,), jnp.float32),
        mesh=mesh,
        compiler_params=pltpu.CompilerParams(
            use_tc_tiling_on_sc=False, needs_layout_passes=False
        ),
        scratch_types=[
            pltpu.VMEM((BPW,), jnp.int32),
            pltpu.VMEM((BPW,), jnp.int32),
            pltpu.VMEM((BPW, D), jnp.float32),
            pltpu.VMEM((BPW, D), jnp.float32),
            pltpu.VMEM((BPW,), jnp.float32),
            pltpu.SemaphoreType.DMA,
            pltpu.SemaphoreType.DMA,
        ],
    )
    return f(user_ids, item_ids, user_table, item_table)


# speedup vs baseline: 1.4952x; 1.4952x over previous
import jax
import jax.numpy as jnp
from jax import lax
from jax.experimental import pallas as pl
from jax.experimental.pallas import tpu as pltpu
from jax.experimental.pallas import tpu_sc as plsc

B = 16384
D = 32
NC = 2
NS = 16
L = 16
NW = NC * NS
BPW = B // NW
CHUNK = 128
NCHUNK = BPW // CHUNK


def _sc_body(uid_hbm, iid_hbm, ut_hbm, it_hbm, out_hbm,
             uidx_v, iidx_v, ub, ib, out_v,
             sem_u0, sem_u1, sem_i0, sem_i1):
    wid = lax.axis_index("s") * NC + lax.axis_index("c")
    base = wid * BPW
    pltpu.sync_copy(uid_hbm.at[pl.ds(base, BPW)], uidx_v)
    pltpu.sync_copy(iid_hbm.at[pl.ds(base, BPW)], iidx_v)

    sems_u = (sem_u0, sem_u1)
    sems_i = (sem_i0, sem_i1)

    def start(c):
        buf = c % 2

        def issue(g, carry):
            uvec = uidx_v[pl.ds(c * CHUNK + g * L, L)]
            ivec = iidx_v[pl.ds(c * CHUNK + g * L, L)]
            for j in range(L):
                pltpu.make_async_copy(
                    ut_hbm.at[pl.ds(uvec[j], 1), :],
                    ub.at[buf].at[pl.ds(g * L + j, 1), :],
                    sems_u[buf],
                ).start()
                pltpu.make_async_copy(
                    it_hbm.at[pl.ds(ivec[j], 1), :],
                    ib.at[buf].at[pl.ds(g * L + j, 1), :],
                    sems_i[buf],
                ).start()
            return carry

        lax.fori_loop(0, CHUNK // L, issue, 0)

    def drain(c):
        buf = c % 2

        def wait_row(r, carry):
            pltpu.make_async_copy(
                ut_hbm.at[pl.ds(0, 1), :],
                ub.at[buf].at[pl.ds(r, 1), :],
                sems_u[buf],
            ).wait()
            pltpu.make_async_copy(
                it_hbm.at[pl.ds(0, 1), :],
                ib.at[buf].at[pl.ds(r, 1), :],
                sems_i[buf],
            ).wait()
            return carry

        lax.fori_loop(0, CHUNK, wait_row, 0)

    lanes = lax.iota(jnp.int32, L)

    start(0)
    for c in range(NCHUNK):
        if c + 1 < NCHUNK:
            start(c + 1)
        drain(c)
        buf = c % 2

        def group(g, carry, _buf=buf, _c=c):
            rows = g * L + lanes
            acc = jnp.zeros((L,), jnp.float32)
            for d0 in range(D):
                dcol = (d0 + lanes) & (D - 1)
                uc = plsc.load_gather(ub.at[_buf], [rows, dcol])
                ic = plsc.load_gather(ib.at[_buf], [rows, dcol])
                acc = acc + uc * ic
            out_v[pl.ds(_c * CHUNK + g * L, L)] = acc
            return carry

        lax.fori_loop(0, CHUNK // L, group, 0)

    pltpu.sync_copy(out_v, out_hbm.at[pl.ds(base, BPW)])


@jax.jit
def kernel(user_ids, item_ids, user_table, item_table):
    mesh = plsc.VectorSubcoreMesh(core_axis_name="c", subcore_axis_name="s")
    f = pl.kernel(
        _sc_body,
        out_type=jax.ShapeDtypeStruct((B,), jnp.float32),
        mesh=mesh,
        compiler_params=pltpu.CompilerParams(needs_layout_passes=False),
        scratch_types=[
            pltpu.VMEM((BPW,), jnp.int32),
            pltpu.VMEM((BPW,), jnp.int32),
            pltpu.VMEM((2, CHUNK, D), jnp.float32),
            pltpu.VMEM((2, CHUNK, D), jnp.float32),
            pltpu.VMEM((BPW,), jnp.float32),
            pltpu.SemaphoreType.DMA,
            pltpu.SemaphoreType.DMA,
            pltpu.SemaphoreType.DMA,
            pltpu.SemaphoreType.DMA,
        ],
    )
    return f(user_ids, item_ids, user_table, item_table)


# speedup vs baseline: 3.8569x; 2.5795x over previous
import jax
import jax.numpy as jnp
from jax import lax
from jax.experimental import pallas as pl
from jax.experimental.pallas import tpu as pltpu
from jax.experimental.pallas import tpu_sc as plsc

B = 16384
D = 32
NC = 2
NS = 16
L = 16
NW = NC * NS
BPW = B // NW
RING = 4
BLK = 128


def _sc_body(uid_hbm, iid_hbm, ut_hbm, it_hbm, out_hbm,
             uidx_v, iidx_v, ub, ib, out_v, sems_u, sems_i):
    wid = lax.axis_index("s") * NC + lax.axis_index("c")
    base = wid * BPW
    pltpu.sync_copy(uid_hbm.at[pl.ds(base, BPW)], uidx_v)
    pltpu.sync_copy(iid_hbm.at[pl.ds(base, BPW)], iidx_v)

    lanes = lax.iota(jnp.int32, L)
    NG = BPW // L

    def issue(uid, iid, slot):
        ublk = pl.multiple_of(uid & ~(BLK - 1), BLK)
        iblk = pl.multiple_of(iid & ~(BLK - 1), BLK)
        pltpu.make_async_copy(
            ut_hbm.at[:, pl.ds(ublk, BLK)], ub.at[slot], sems_u.at[slot]
        ).start()
        pltpu.make_async_copy(
            it_hbm.at[:, pl.ds(iblk, BLK)], ib.at[slot], sems_i.at[slot]
        ).start()

    def wait(slot):
        pltpu.make_async_copy(
            ut_hbm.at[:, pl.ds(0, BLK)], ub.at[slot], sems_u.at[slot]
        ).wait()
        pltpu.make_async_copy(
            it_hbm.at[:, pl.ds(0, BLK)], ib.at[slot], sems_i.at[slot]
        ).wait()

    def extract_dot(uid, iid, slot):
        ucols = jnp.zeros((L,), jnp.int32) + (uid & (BLK - 1))
        icols = jnp.zeros((L,), jnp.int32) + (iid & (BLK - 1))
        u0 = plsc.load_gather(ub.at[slot], [lanes, ucols])
        u1 = plsc.load_gather(ub.at[slot], [lanes + L, ucols])
        i0 = plsc.load_gather(ib.at[slot], [lanes, icols])
        i1 = plsc.load_gather(ib.at[slot], [lanes + L, icols])
        return jnp.sum(u0 * i0 + u1 * i1)

    vu0 = uidx_v[pl.ds(0, L)]
    vi0 = iidx_v[pl.ds(0, L)]
    for s in range(RING):
        issue(vu0[s], vi0[s], s)

    def step(g, carry):
        vu = uidx_v[pl.ds(g * L, L)]
        vi = iidx_v[pl.ds(g * L, L)]
        nxt = jnp.minimum(g + 1, NG - 1) * L
        vun = uidx_v[pl.ds(nxt, L)]
        vin = iidx_v[pl.ds(nxt, L)]
        acc = jnp.zeros((L,), jnp.float32)
        for j in range(L):
            slot = j % RING
            wait(slot)
            dot = extract_dot(vu[j], vi[j], slot)
            if j < L - RING:
                issue(vu[j + RING], vi[j + RING], slot)
            else:

                @pl.when(g + 1 < NG)
                def _(_j=j):
                    issue(vun[_j + RING - L], vin[_j + RING - L], slot)

            acc = jnp.where(lanes == j, dot, acc)
        out_v[pl.ds(g * L, L)] = acc
        return carry

    lax.fori_loop(0, NG, step, 0)
    pltpu.sync_copy(out_v, out_hbm.at[pl.ds(base, BPW)])


@jax.jit
def kernel(user_ids, item_ids, user_table, item_table):
    mesh = plsc.VectorSubcoreMesh(core_axis_name="c", subcore_axis_name="s")
    f = pl.kernel(
        _sc_body,
        out_type=jax.ShapeDtypeStruct((B,), jnp.float32),
        mesh=mesh,
        compiler_params=pltpu.CompilerParams(needs_layout_passes=False),
        scratch_types=[
            pltpu.VMEM((BPW,), jnp.int32),
            pltpu.VMEM((BPW,), jnp.int32),
            pltpu.VMEM((RING, D, BLK), jnp.float32),
            pltpu.VMEM((RING, D, BLK), jnp.float32),
            pltpu.VMEM((BPW,), jnp.float32),
            pltpu.SemaphoreType.DMA((RING,)),
            pltpu.SemaphoreType.DMA((RING,)),
        ],
    )
    return f(user_ids, item_ids, user_table.T, item_table.T)


# speedup vs baseline: 3.9005x; 1.0113x over previous
import jax
import jax.numpy as jnp
from jax import lax
from jax.experimental import pallas as pl
from jax.experimental.pallas import tpu as pltpu
from jax.experimental.pallas import tpu_sc as plsc

B = 16384
D = 32
NC = 2
NS = 16
L = 16
NW = NC * NS
BPW = B // NW
RING = 8
BLK = 128


def _sc_body(uid_hbm, iid_hbm, ut_hbm, it_hbm, out_hbm,
             uidx_v, iidx_v, ub, ib, out_v, sems_u, sems_i):
    wid = lax.axis_index("s") * NC + lax.axis_index("c")
    base = wid * BPW
    pltpu.sync_copy(uid_hbm.at[pl.ds(base, BPW)], uidx_v)
    pltpu.sync_copy(iid_hbm.at[pl.ds(base, BPW)], iidx_v)

    lanes = lax.iota(jnp.int32, L)
    NG = BPW // L

    def issue(uid, iid, slot):
        ublk = pl.multiple_of(uid & ~(BLK - 1), BLK)
        iblk = pl.multiple_of(iid & ~(BLK - 1), BLK)
        pltpu.make_async_copy(
            ut_hbm.at[:, pl.ds(ublk, BLK)], ub.at[slot], sems_u.at[slot]
        ).start()
        pltpu.make_async_copy(
            it_hbm.at[:, pl.ds(iblk, BLK)], ib.at[slot], sems_i.at[slot]
        ).start()

    def wait(slot):
        pltpu.make_async_copy(
            ut_hbm.at[:, pl.ds(0, BLK)], ub.at[slot], sems_u.at[slot]
        ).wait()
        pltpu.make_async_copy(
            it_hbm.at[:, pl.ds(0, BLK)], ib.at[slot], sems_i.at[slot]
        ).wait()

    def extract_dot(uid, iid, slot):
        ucols = jnp.zeros((L,), jnp.int32) + (uid & (BLK - 1))
        icols = jnp.zeros((L,), jnp.int32) + (iid & (BLK - 1))
        u0 = plsc.load_gather(ub.at[slot], [lanes, ucols])
        u1 = plsc.load_gather(ub.at[slot], [lanes + L, ucols])
        i0 = plsc.load_gather(ib.at[slot], [lanes, icols])
        i1 = plsc.load_gather(ib.at[slot], [lanes + L, icols])
        return jnp.sum(u0 * i0 + u1 * i1)

    vu0 = uidx_v[pl.ds(0, L)]
    vi0 = iidx_v[pl.ds(0, L)]
    for s in range(RING):
        issue(vu0[s], vi0[s], s)

    def step(g, carry):
        vu = uidx_v[pl.ds(g * L, L)]
        vi = iidx_v[pl.ds(g * L, L)]
        nxt = jnp.minimum(g + 1, NG - 1) * L
        vun = uidx_v[pl.ds(nxt, L)]
        vin = iidx_v[pl.ds(nxt, L)]
        acc = jnp.zeros((L,), jnp.float32)
        for j in range(L):
            slot = j % RING
            wait(slot)
            dot = extract_dot(vu[j], vi[j], slot)
            if j < L - RING:
                issue(vu[j + RING], vi[j + RING], slot)
            else:

                @pl.when(g + 1 < NG)
                def _(_j=j):
                    issue(vun[_j + RING - L], vin[_j + RING - L], slot)

            acc = jnp.where(lanes == j, dot, acc)
        out_v[pl.ds(g * L, L)] = acc
        return carry

    lax.fori_loop(0, NG, step, 0)
    pltpu.sync_copy(out_v, out_hbm.at[pl.ds(base, BPW)])


@jax.jit
def kernel(user_ids, item_ids, user_table, item_table):
    mesh = plsc.VectorSubcoreMesh(core_axis_name="c", subcore_axis_name="s")
    f = pl.kernel(
        _sc_body,
        out_type=jax.ShapeDtypeStruct((B,), jnp.float32),
        mesh=mesh,
        compiler_params=pltpu.CompilerParams(needs_layout_passes=False),
        scratch_types=[
            pltpu.VMEM((BPW,), jnp.int32),
            pltpu.VMEM((BPW,), jnp.int32),
            pltpu.VMEM((RING, D, BLK), jnp.float32),
            pltpu.VMEM((RING, D, BLK), jnp.float32),
            pltpu.VMEM((BPW,), jnp.float32),
            pltpu.SemaphoreType.DMA((RING,)),
            pltpu.SemaphoreType.DMA((RING,)),
        ],
    )
    return f(user_ids, item_ids, user_table.T, item_table.T)


# speedup vs baseline: 3.9022x; 1.0005x over previous
import jax
import jax.numpy as jnp
from jax import lax
from jax.experimental import pallas as pl
from jax.experimental.pallas import tpu as pltpu
from jax.experimental.pallas import tpu_sc as plsc

B = 16384
D = 32
NC = 2
NS = 16
L = 16
NW = NC * NS
BPW = B // NW
RING = 8
BLK = 128


def _sc_body(uid_hbm, iid_hbm, ut_hbm, it_hbm, out_hbm,
             uidx_v, iidx_v, ub, ib, out_v, sems_u, sems_i):
    wid = lax.axis_index("s") * NC + lax.axis_index("c")
    base = wid * BPW
    pltpu.sync_copy(uid_hbm.at[pl.ds(base, BPW)], uidx_v)
    pltpu.sync_copy(iid_hbm.at[pl.ds(base, BPW)], iidx_v)

    lanes = lax.iota(jnp.int32, L)
    NG = BPW // L

    nu = ut_hbm.shape[1]
    ut_slabs = ut_hbm.reshape(D // 8, 8, nu)
    it_slabs = it_hbm.reshape(D // 8, 8, nu)

    def issue(uid, iid, slot):
        ublk = pl.multiple_of(uid & ~(BLK - 1), BLK)
        iblk = pl.multiple_of(iid & ~(BLK - 1), BLK)
        for j4 in range(D // 8):
            pltpu.make_async_copy(
                ut_slabs.at[j4, :, pl.ds(ublk, BLK)],
                ub.at[slot].at[pl.ds(j4 * 8, 8), :],
                sems_u.at[slot],
            ).start()
            pltpu.make_async_copy(
                it_slabs.at[j4, :, pl.ds(iblk, BLK)],
                ib.at[slot].at[pl.ds(j4 * 8, 8), :],
                sems_i.at[slot],
            ).start()

    def wait(slot):
        pltpu.make_async_copy(
            ut_hbm.at[:, pl.ds(0, BLK)], ub.at[slot], sems_u.at[slot]
        ).wait()
        pltpu.make_async_copy(
            it_hbm.at[:, pl.ds(0, BLK)], ib.at[slot], sems_i.at[slot]
        ).wait()

    def extract_dot(uid, iid, slot):
        ucols = jnp.zeros((L,), jnp.int32) + (uid & (BLK - 1))
        icols = jnp.zeros((L,), jnp.int32) + (iid & (BLK - 1))
        u0 = plsc.load_gather(ub.at[slot], [lanes, ucols])
        u1 = plsc.load_gather(ub.at[slot], [lanes + L, ucols])
        i0 = plsc.load_gather(ib.at[slot], [lanes, icols])
        i1 = plsc.load_gather(ib.at[slot], [lanes + L, icols])
        return jnp.sum(u0 * i0 + u1 * i1)

    vu0 = uidx_v[pl.ds(0, L)]
    vi0 = iidx_v[pl.ds(0, L)]
    for s in range(RING):
        issue(vu0[s], vi0[s], s)

    def step(g, carry):
        vu = uidx_v[pl.ds(g * L, L)]
        vi = iidx_v[pl.ds(g * L, L)]
        nxt = jnp.minimum(g + 1, NG - 1) * L
        vun = uidx_v[pl.ds(nxt, L)]
        vin = iidx_v[pl.ds(nxt, L)]
        acc = jnp.zeros((L,), jnp.float32)
        for j in range(L):
            slot = j % RING
            wait(slot)
            dot = extract_dot(vu[j], vi[j], slot)
            if j < L - RING:
                issue(vu[j + RING], vi[j + RING], slot)
            else:

                @pl.when(g + 1 < NG)
                def _(_j=j):
                    issue(vun[_j + RING - L], vin[_j + RING - L], slot)

            acc = jnp.where(lanes == j, dot, acc)
        out_v[pl.ds(g * L, L)] = acc
        return carry

    lax.fori_loop(0, NG, step, 0)
    pltpu.sync_copy(out_v, out_hbm.at[pl.ds(base, BPW)])


@jax.jit
def kernel(user_ids, item_ids, user_table, item_table):
    mesh = plsc.VectorSubcoreMesh(core_axis_name="c", subcore_axis_name="s")
    f = pl.kernel(
        _sc_body,
        out_type=jax.ShapeDtypeStruct((B,), jnp.float32),
        mesh=mesh,
        compiler_params=pltpu.CompilerParams(needs_layout_passes=False),
        scratch_types=[
            pltpu.VMEM((BPW,), jnp.int32),
            pltpu.VMEM((BPW,), jnp.int32),
            pltpu.VMEM((RING, D, BLK), jnp.float32),
            pltpu.VMEM((RING, D, BLK), jnp.float32),
            pltpu.VMEM((BPW,), jnp.float32),
            pltpu.SemaphoreType.DMA((RING,)),
            pltpu.SemaphoreType.DMA((RING,)),
        ],
    )
    return f(user_ids, item_ids, user_table.T, item_table.T)
